# baseline (speedup 1.0000x reference)
import functools

import jax
import jax.numpy as jnp
from jax import lax
from jax.experimental import pallas as pl
from jax.experimental.pallas import tpu as pltpu
from jax.experimental.pallas import tpu_sc as plsc

NC = 2
NS = 16
L = 16


def _mm_body(x_ref, w_ref, o_ref):
    acc = lax.dot_general(x_ref[...], w_ref[...],
                          (((1,), (1,)), ((), ())),
                          preferred_element_type=jnp.float32)
    hc = acc.shape[1] // 2
    o_ref[0] = acc[:, :hc]
    o_ref[1] = acc[:, hc:]


def _matmul_split(nodes, w):
    n, h = nodes.shape
    hc = h // NC
    bn = 2000
    return pl.pallas_call(
        _mm_body,
        grid=(n // bn,),
        in_specs=[
            pl.BlockSpec((bn, h), lambda i: (i, 0)),
            pl.BlockSpec((h, h), lambda i: (0, 0)),
        ],
        out_specs=pl.BlockSpec((NC, bn, hc), lambda i: (0, i, 0)),
        out_shape=jax.ShapeDtypeStruct((NC, n, hc), jnp.float32),
    )(nodes, w)


B = 125
SG = 10
EC = 50


@functools.cache
def _make_sc(n, h, a, e):
    hc = h // NC
    pw = a // NS
    nch = pw // B
    ept = e // NS
    mesh = plsc.VectorSubcoreMesh(core_axis_name="c", subcore_axis_name="s")

    @functools.partial(
        pl.kernel,
        out_type=[
            jax.ShapeDtypeStruct((a, h), jnp.float32),
            jax.ShapeDtypeStruct((e, h), jnp.float32),
        ],
        mesh=mesh,
        compiler_params=pltpu.CompilerParams(use_tc_tiling_on_sc=False),
        scratch_types=[
            pltpu.VMEM((SG, B), jnp.int32),
            pltpu.VMEM((SG, B), jnp.int32),
            pltpu.VMEM((3, B, hc), jnp.float32),
            pltpu.VMEM((B, L), jnp.float32),
            pltpu.VMEM_SHARED((e, hc), jnp.float32),
            pltpu.VMEM_SHARED((e, L), jnp.float32),
            pltpu.SemaphoreType.DMA,
            pltpu.SemaphoreType.DMA,
            pltpu.SemaphoreType.DMA,
            pltpu.SemaphoreType.DMA,
        ],
    )
    def sc_fn(table_hbm, gidx_hbm, eid_hbm, out1_hbm, out2_hbm,
              gidx_v, eid_v, rows_v, ones_v,
              agg_sh, deg_sh, sem_g, sem_s, sem_w, sem_d):
        c = lax.axis_index("c")
        s = lax.axis_index("s")
        e0 = s * ept
        a0 = s * pw

        _p0 = jax.named_scope("p0_zero"); _p0.__enter__()
        def _init_ones(i, _):
            ones_v[i] = jnp.ones((L,), jnp.float32)
            return 0
        lax.fori_loop(0, B, _init_ones, 0)

        def _zero_row(i, _):
            z = jnp.zeros((L,), jnp.float32)
            for k in range(hc // L):
                rows_v[0, i, pl.ds(k * L, L)] = z
            return 0
        lax.fori_loop(0, EC, _zero_row, 0)

        def _zero_sh(q, _):
            pltpu.sync_copy(rows_v.at[0].at[pl.ds(0, EC)],
                            agg_sh.at[pl.ds(e0 + q * EC, EC)])
            pltpu.sync_copy(rows_v.at[0].at[pl.ds(0, EC), pl.ds(0, L)],
                            deg_sh.at[pl.ds(e0 + q * EC, EC)])
            return 0
        lax.fori_loop(0, ept // EC, _zero_sh, 0)
        plsc.subcore_barrier()
        _p0.__exit__(None, None, None)
        _p1 = jax.named_scope("p1_accum"); _p1.__enter__()

        def _grp1(q, _):
            r0 = s * nch + q * SG
            pltpu.sync_copy(gidx_hbm.at[c, pl.ds(r0, SG)], gidx_v)
            pltpu.sync_copy(eid_hbm.at[pl.ds(r0, SG)], eid_v)
            pltpu.async_copy(table_hbm.at[gidx_v.at[0]], rows_v.at[0], sem_g)
            pltpu.async_copy(table_hbm.at[gidx_v.at[1]], rows_v.at[1], sem_g)

            def _chunk(j, _):
                cur = lax.rem(j, 3)
                pltpu.make_async_copy(
                    table_hbm.at[gidx_v.at[j]], rows_v.at[cur], sem_g).wait()

                @pl.when(j >= 1)
                def _():
                    pltpu.make_async_copy(
                        rows_v.at[lax.rem(j + 2, 3)],
                        agg_sh.at[eid_v.at[j - 1]], sem_s).wait()

                @pl.when(j + 2 < SG)
                def _():
                    pltpu.async_copy(
                        table_hbm.at[gidx_v.at[j + 2]],
                        rows_v.at[lax.rem(j + 2, 3)], sem_g)

                pltpu.async_copy(
                    rows_v.at[cur], agg_sh.at[eid_v.at[j]], sem_s, add=True)
                pltpu.async_copy(ones_v, deg_sh.at[eid_v.at[j]], sem_d,
                                 add=True)
                return 0
            lax.fori_loop(0, SG, _chunk, 0)
            pltpu.make_async_copy(
                rows_v.at[(SG - 1) % 3], agg_sh.at[eid_v.at[SG - 1]],
                sem_s).wait()

            def _drain(j, _):
                pltpu.make_async_copy(
                    ones_v, deg_sh.at[eid_v.at[0]], sem_d).wait()
                return 0
            lax.fori_loop(0, SG, _drain, 0)
            return 0
        lax.fori_loop(0, nch // SG, _grp1, 0)
        plsc.subcore_barrier()
        _p1.__exit__(None, None, None)
        _p2 = jax.named_scope("p2_scale"); _p2.__enter__()

        ng2 = ept // EC

        def _stage2(q, buf):
            eq = e0 + q * EC
            pltpu.async_copy(agg_sh.at[pl.ds(eq, EC)], rows_v.at[buf].at[pl.ds(0, EC)], sem_g)
            pltpu.async_copy(deg_sh.at[pl.ds(eq, EC)], ones_v.at[pl.ds(buf * EC, EC)], sem_g)

        def _unstage2(q, buf):
            eq = e0 + q * EC
            pltpu.make_async_copy(agg_sh.at[pl.ds(eq, EC)], rows_v.at[buf].at[pl.ds(0, EC)], sem_g).wait()
            pltpu.make_async_copy(deg_sh.at[pl.ds(eq, EC)], ones_v.at[pl.ds(buf * EC, EC)], sem_g).wait()

        def _wrwait2(q, buf):
            eq = e0 + q * EC
            pltpu.make_async_copy(rows_v.at[buf].at[pl.ds(0, EC)], agg_sh.at[pl.ds(eq, EC)], sem_s).wait()
            pltpu.make_async_copy(rows_v.at[buf].at[pl.ds(0, EC)], out2_hbm.at[pl.ds(eq, EC), pl.ds(c * hc, hc)], sem_w).wait()

        _stage2(0, 0)

        def _grp2(q, _):
            eq = e0 + q * EC
            cur = lax.rem(q, 2)
            nxt = 1 - cur
            _unstage2(q, cur)

            @pl.when(q >= 1)
            def _():
                _wrwait2(q - 1, nxt)

            @pl.when(q + 1 < ng2)
            def _():
                _stage2(q + 1, nxt)

            def _scale_row(i, _):
                inv = 1.0 / ones_v[cur * EC + i]
                for k in range(hc // L):
                    sl = pl.ds(k * L, L)
                    rows_v[cur, i, sl] = rows_v[cur, i, sl] * inv
                return 0
            lax.fori_loop(0, EC, _scale_row, 0)
            pltpu.async_copy(rows_v.at[cur].at[pl.ds(0, EC)], agg_sh.at[pl.ds(eq, EC)], sem_s)
            pltpu.async_copy(rows_v.at[cur].at[pl.ds(0, EC)], out2_hbm.at[pl.ds(eq, EC), pl.ds(c * hc, hc)], sem_w)
            return 0
        lax.fori_loop(0, ng2, _grp2, 0)
        _wrwait2(ng2 - 1, lax.rem(ng2 - 1, 2))
        plsc.subcore_barrier()
        _p2.__exit__(None, None, None)
        _p3 = jax.named_scope("p3_back"); _p3.__enter__()

        def _grp3(q, _):
            pltpu.sync_copy(eid_hbm.at[pl.ds(s * nch + q * SG, SG)], eid_v)
            pltpu.async_copy(agg_sh.at[eid_v.at[0]], rows_v.at[0], sem_g)
            pltpu.async_copy(agg_sh.at[eid_v.at[1]], rows_v.at[1], sem_g)

            def _chunk(j, _):
                cur = lax.rem(j, 3)
                pltpu.make_async_copy(
                    agg_sh.at[eid_v.at[j]], rows_v.at[cur], sem_g).wait()

                @pl.when(j >= 1)
                def _():
                    pltpu.make_async_copy(
                        rows_v.at[lax.rem(j + 2, 3)],
                        out1_hbm.at[pl.ds(a0 + (q * SG + j - 1) * B, B),
                                    pl.ds(c * hc, hc)],
                        sem_w).wait()

                @pl.when(j + 2 < SG)
                def _():
                    pltpu.async_copy(
                        agg_sh.at[eid_v.at[j + 2]],
                        rows_v.at[lax.rem(j + 2, 3)], sem_g)

                pltpu.async_copy(
                    rows_v.at[cur],
                    out1_hbm.at[pl.ds(a0 + (q * SG + j) * B, B),
                                pl.ds(c * hc, hc)], sem_w)
                return 0
            lax.fori_loop(0, SG, _chunk, 0)
            pltpu.make_async_copy(
                rows_v.at[(SG - 1) % 3],
                out1_hbm.at[pl.ds(a0 + (q * SG + SG - 1) * B, B),
                            pl.ds(c * hc, hc)],
                sem_w).wait()
            return 0
        lax.fori_loop(0, nch // SG, _grp3, 0)
        _p3.__exit__(None, None, None)

    return sc_fn


def kernel(nodes_representations, hyperedge_arg_node_idxs,
           unq_hyperedge_type_reprs, hyperedge_type_name_unq_idxs,
           unq_hyperedge_arg_name_reprs, hyperedge_arg_name_unq_idxs,
           hyperedge_arg_to_edge_id, num_edges, W):
    n, h = nodes_representations.shape
    a = hyperedge_arg_node_idxs.shape[0]
    e = hyperedge_type_name_unq_idxs.shape[0]

    table = _matmul_split(nodes_representations, W).reshape(NC * n, h // NC)

    idx = hyperedge_arg_node_idxs.astype(jnp.int32)
    gidx = jnp.stack([idx, idx + n]).reshape(NC, a // B, B)
    eid2 = hyperedge_arg_to_edge_id.astype(jnp.int32).reshape(a // B, B)

    out1, out2 = _make_sc(n, h, a, e)(table, gidx, eid2)
    return out1, out2


# speedup vs baseline: 1.0046x; 1.0046x over previous
import functools

import jax
import jax.numpy as jnp
from jax import lax
from jax.experimental import pallas as pl
from jax.experimental.pallas import tpu as pltpu
from jax.experimental.pallas import tpu_sc as plsc

NC = 2
NS = 16
L = 16


def _mm_body(x_ref, w_ref, o_ref):
    acc = lax.dot_general(x_ref[...], w_ref[...],
                          (((1,), (1,)), ((), ())),
                          preferred_element_type=jnp.float32)
    hc = acc.shape[1] // 2
    o_ref[0] = acc[:, :hc]
    o_ref[1] = acc[:, hc:]


def _matmul_split(nodes, w):
    n, h = nodes.shape
    hc = h // NC
    bn = 2000
    return pl.pallas_call(
        _mm_body,
        grid=(n // bn,),
        in_specs=[
            pl.BlockSpec((bn, h), lambda i: (i, 0)),
            pl.BlockSpec((h, h), lambda i: (0, 0)),
        ],
        out_specs=pl.BlockSpec((NC, bn, hc), lambda i: (0, i, 0)),
        out_shape=jax.ShapeDtypeStruct((NC, n, hc), jnp.float32),
    )(nodes, w)


B = 125
SG = 10
EC = 50


@functools.cache
def _make_sc(n, h, a, e):
    hc = h // NC
    pw = a // NS
    nch = pw // B
    ept = e // NS
    mesh = plsc.VectorSubcoreMesh(core_axis_name="c", subcore_axis_name="s")

    @functools.partial(
        pl.kernel,
        out_type=[
            jax.ShapeDtypeStruct((a, h), jnp.float32),
            jax.ShapeDtypeStruct((e, h), jnp.float32),
        ],
        mesh=mesh,
        compiler_params=pltpu.CompilerParams(use_tc_tiling_on_sc=False, needs_layout_passes=False),
        scratch_types=[
            pltpu.VMEM((SG, B), jnp.int32),
            pltpu.VMEM((SG, B), jnp.int32),
            pltpu.VMEM((3, B, hc), jnp.float32),
            pltpu.VMEM((B, 8), jnp.float32),
            pltpu.VMEM((2 * EC, 8), jnp.float32),
            pltpu.VMEM_SHARED((e, hc), jnp.float32),
            pltpu.VMEM_SHARED((e, 8), jnp.float32),
            pltpu.SemaphoreType.DMA,
            pltpu.SemaphoreType.DMA,
            pltpu.SemaphoreType.DMA,
            pltpu.SemaphoreType.DMA,
        ],
    )
    def sc_fn(table_hbm, gidx_hbm, eid_hbm, ones_hbm, out1_hbm, out2_hbm,
              gidx_v, eid_v, rows_v, ones_v, dstg_v,
              agg_sh, deg_sh, sem_g, sem_s, sem_w, sem_d):
        c = lax.axis_index("c")
        s = lax.axis_index("s")
        e0 = s * ept
        a0 = s * pw

        _p0 = jax.named_scope("p0_zero"); _p0.__enter__()
        pltpu.sync_copy(ones_hbm, ones_v)

        def _zero_row(i, _):
            z = jnp.zeros((L,), jnp.float32)
            for k in range(hc // L):
                rows_v[0, i, pl.ds(k * L, L)] = z
            return 0
        lax.fori_loop(0, EC, _zero_row, 0)

        def _zero_sh(q, _):
            pltpu.sync_copy(rows_v.at[0].at[pl.ds(0, EC)],
                            agg_sh.at[pl.ds(e0 + q * EC, EC)])
            pltpu.sync_copy(rows_v.at[0].at[pl.ds(0, EC), pl.ds(0, 8)],
                            deg_sh.at[pl.ds(e0 + q * EC, EC)])
            return 0
        lax.fori_loop(0, ept // EC, _zero_sh, 0)
        plsc.subcore_barrier()
        _p0.__exit__(None, None, None)
        _p1 = jax.named_scope("p1_accum"); _p1.__enter__()

        def _grp1(q, _):
            r0 = s * nch + q * SG
            pltpu.sync_copy(gidx_hbm.at[c, pl.ds(r0, SG)], gidx_v)
            pltpu.sync_copy(eid_hbm.at[pl.ds(r0, SG)], eid_v)
            pltpu.async_copy(table_hbm.at[gidx_v.at[0]], rows_v.at[0], sem_g)
            pltpu.async_copy(table_hbm.at[gidx_v.at[1]], rows_v.at[1], sem_g)

            def _chunk(j, _):
                cur = lax.rem(j, 3)
                pltpu.make_async_copy(
                    table_hbm.at[gidx_v.at[j]], rows_v.at[cur], sem_g).wait()

                @pl.when(j >= 1)
                def _():
                    pltpu.make_async_copy(
                        rows_v.at[lax.rem(j + 2, 3)],
                        agg_sh.at[eid_v.at[j - 1]], sem_s).wait()

                @pl.when(j + 2 < SG)
                def _():
                    pltpu.async_copy(
                        table_hbm.at[gidx_v.at[j + 2]],
                        rows_v.at[lax.rem(j + 2, 3)], sem_g)

                pltpu.async_copy(
                    rows_v.at[cur], agg_sh.at[eid_v.at[j]], sem_s, add=True)
                pltpu.async_copy(ones_v, deg_sh.at[eid_v.at[j]], sem_d,
                                 add=True)
                return 0
            lax.fori_loop(0, SG, _chunk, 0)
            pltpu.make_async_copy(
                rows_v.at[(SG - 1) % 3], agg_sh.at[eid_v.at[SG - 1]],
                sem_s).wait()

            def _drain(j, _):
                pltpu.make_async_copy(
                    ones_v, deg_sh.at[eid_v.at[0]], sem_d).wait()
                return 0
            lax.fori_loop(0, SG, _drain, 0)
            return 0
        lax.fori_loop(0, nch // SG, _grp1, 0)
        plsc.subcore_barrier()
        _p1.__exit__(None, None, None)
        _p2 = jax.named_scope("p2_scale"); _p2.__enter__()

        ng2 = ept // EC

        def _stage2(q, buf):
            eq = e0 + q * EC
            pltpu.async_copy(agg_sh.at[pl.ds(eq, EC)], rows_v.at[buf].at[pl.ds(0, EC)], sem_g)
            pltpu.async_copy(deg_sh.at[pl.ds(eq, EC)], dstg_v.at[pl.ds(buf * EC, EC)], sem_g)

        def _unstage2(q, buf):
            eq = e0 + q * EC
            pltpu.make_async_copy(agg_sh.at[pl.ds(eq, EC)], rows_v.at[buf].at[pl.ds(0, EC)], sem_g).wait()
            pltpu.make_async_copy(deg_sh.at[pl.ds(eq, EC)], dstg_v.at[pl.ds(buf * EC, EC)], sem_g).wait()

        def _wrwait2(q, buf):
            eq = e0 + q * EC
            pltpu.make_async_copy(rows_v.at[buf].at[pl.ds(0, EC)], agg_sh.at[pl.ds(eq, EC)], sem_s).wait()
            pltpu.make_async_copy(rows_v.at[buf].at[pl.ds(0, EC)], out2_hbm.at[pl.ds(eq, EC), pl.ds(c * hc, hc)], sem_w).wait()

        _stage2(0, 0)

        def _grp2(q, _):
            eq = e0 + q * EC
            cur = lax.rem(q, 2)
            nxt = 1 - cur
            _unstage2(q, cur)

            @pl.when(q >= 1)
            def _():
                _wrwait2(q - 1, nxt)

            @pl.when(q + 1 < ng2)
            def _():
                _stage2(q + 1, nxt)

            def _scale_row(i, _):
                ridx = jnp.full((L,), cur * EC + i, jnp.int32)
                cidx = jnp.zeros((L,), jnp.int32)
                inv = 1.0 / plsc.load_gather(dstg_v, [ridx, cidx])
                for k in range(hc // L):
                    sl = pl.ds(k * L, L)
                    rows_v[cur, i, sl] = rows_v[cur, i, sl] * inv
                return 0
            lax.fori_loop(0, EC, _scale_row, 0)
            pltpu.async_copy(rows_v.at[cur].at[pl.ds(0, EC)], agg_sh.at[pl.ds(eq, EC)], sem_s)
            pltpu.async_copy(rows_v.at[cur].at[pl.ds(0, EC)], out2_hbm.at[pl.ds(eq, EC), pl.ds(c * hc, hc)], sem_w)
            return 0
        lax.fori_loop(0, ng2, _grp2, 0)
        _wrwait2(ng2 - 1, lax.rem(ng2 - 1, 2))
        plsc.subcore_barrier()
        _p2.__exit__(None, None, None)
        _p3 = jax.named_scope("p3_back"); _p3.__enter__()

        def _grp3(q, _):
            pltpu.sync_copy(eid_hbm.at[pl.ds(s * nch + q * SG, SG)], eid_v)
            pltpu.async_copy(agg_sh.at[eid_v.at[0]], rows_v.at[0], sem_g)
            pltpu.async_copy(agg_sh.at[eid_v.at[1]], rows_v.at[1], sem_g)

            def _chunk(j, _):
                cur = lax.rem(j, 3)
                pltpu.make_async_copy(
                    agg_sh.at[eid_v.at[j]], rows_v.at[cur], sem_g).wait()

                @pl.when(j >= 1)
                def _():
                    pltpu.make_async_copy(
                        rows_v.at[lax.rem(j + 2, 3)],
                        out1_hbm.at[pl.ds(a0 + (q * SG + j - 1) * B, B),
                                    pl.ds(c * hc, hc)],
                        sem_w).wait()

                @pl.when(j + 2 < SG)
                def _():
                    pltpu.async_copy(
                        agg_sh.at[eid_v.at[j + 2]],
                        rows_v.at[lax.rem(j + 2, 3)], sem_g)

                pltpu.async_copy(
                    rows_v.at[cur],
                    out1_hbm.at[pl.ds(a0 + (q * SG + j) * B, B),
                                pl.ds(c * hc, hc)], sem_w)
                return 0
            lax.fori_loop(0, SG, _chunk, 0)
            pltpu.make_async_copy(
                rows_v.at[(SG - 1) % 3],
                out1_hbm.at[pl.ds(a0 + (q * SG + SG - 1) * B, B),
                            pl.ds(c * hc, hc)],
                sem_w).wait()
            return 0
        lax.fori_loop(0, nch // SG, _grp3, 0)
        _p3.__exit__(None, None, None)

    return sc_fn


def kernel(nodes_representations, hyperedge_arg_node_idxs,
           unq_hyperedge_type_reprs, hyperedge_type_name_unq_idxs,
           unq_hyperedge_arg_name_reprs, hyperedge_arg_name_unq_idxs,
           hyperedge_arg_to_edge_id, num_edges, W):
    n, h = nodes_representations.shape
    a = hyperedge_arg_node_idxs.shape[0]
    e = hyperedge_type_name_unq_idxs.shape[0]

    table = _matmul_split(nodes_representations, W).reshape(NC * n, h // NC)

    idx = hyperedge_arg_node_idxs.astype(jnp.int32)
    gidx = jnp.stack([idx, idx + n]).reshape(NC, a // B, B)
    eid2 = hyperedge_arg_to_edge_id.astype(jnp.int32).reshape(a // B, B)

    ones8 = jnp.ones((B, 8), jnp.float32)
    out1, out2 = _make_sc(n, h, a, e)(table, gidx, eid2, ones8)
    return out1, out2


# speedup vs baseline: 1.0113x; 1.0067x over previous
import functools

import jax
import jax.numpy as jnp
from jax import lax
from jax.experimental import pallas as pl
from jax.experimental.pallas import tpu as pltpu
from jax.experimental.pallas import tpu_sc as plsc

NC = 2
NS = 16
L = 16


def _mm_body(x_ref, w_ref, o_ref):
    acc = lax.dot_general(x_ref[...], w_ref[...],
                          (((1,), (1,)), ((), ())),
                          preferred_element_type=jnp.float32)
    hc = acc.shape[1] // 2
    o_ref[0] = acc[:, :hc]
    o_ref[1] = acc[:, hc:]


def _matmul_split(nodes, w):
    n, h = nodes.shape
    hc = h // NC
    bn = 2000
    return pl.pallas_call(
        _mm_body,
        grid=(n // bn,),
        in_specs=[
            pl.BlockSpec((bn, h), lambda i: (i, 0)),
            pl.BlockSpec((h, h), lambda i: (0, 0)),
        ],
        out_specs=pl.BlockSpec((NC, bn, hc), lambda i: (0, i, 0)),
        out_shape=jax.ShapeDtypeStruct((NC, n, hc), jnp.float32),
    )(nodes, w)


B = 125
SG = 10
EC = 50


@functools.cache
def _make_sc(n, h, a, e):
    hc = h // NC
    pw = a // NS
    nch = pw // B
    ept = e // NS
    mesh = plsc.VectorSubcoreMesh(core_axis_name="c", subcore_axis_name="s")

    @functools.partial(
        pl.kernel,
        out_type=[
            jax.ShapeDtypeStruct((a, h), jnp.float32),
            jax.ShapeDtypeStruct((e, h), jnp.float32),
        ],
        mesh=mesh,
        compiler_params=pltpu.CompilerParams(use_tc_tiling_on_sc=False, needs_layout_passes=False),
        scratch_types=[
            pltpu.VMEM((SG, B), jnp.int32),
            pltpu.VMEM((SG, B), jnp.int32),
            pltpu.VMEM((4, B, hc), jnp.float32),
            pltpu.VMEM((B, 8), jnp.float32),
            pltpu.VMEM((2 * EC, 8), jnp.float32),
            pltpu.VMEM_SHARED((e, hc), jnp.float32),
            pltpu.VMEM_SHARED((e, 8), jnp.float32),
            pltpu.SemaphoreType.DMA,
            pltpu.SemaphoreType.DMA,
            pltpu.SemaphoreType.DMA,
            pltpu.SemaphoreType.DMA,
        ],
    )
    def sc_fn(table_hbm, gidx_hbm, eid_hbm, ones_hbm, out1_hbm, out2_hbm,
              gidx_v, eid_v, rows_v, ones_v, dstg_v,
              agg_sh, deg_sh, sem_g, sem_s, sem_w, sem_d):
        c = lax.axis_index("c")
        s = lax.axis_index("s")
        e0 = s * ept
        a0 = s * pw

        _p0 = jax.named_scope("p0_zero"); _p0.__enter__()
        pltpu.sync_copy(ones_hbm, ones_v)

        def _zero_row(i, _):
            z = jnp.zeros((L,), jnp.float32)
            for k in range(hc // L):
                rows_v[0, i, pl.ds(k * L, L)] = z
            return 0
        lax.fori_loop(0, EC, _zero_row, 0)

        def _zero_sh(q, _):
            pltpu.sync_copy(rows_v.at[0].at[pl.ds(0, EC)],
                            agg_sh.at[pl.ds(e0 + q * EC, EC)])
            pltpu.sync_copy(rows_v.at[0].at[pl.ds(0, EC), pl.ds(0, 8)],
                            deg_sh.at[pl.ds(e0 + q * EC, EC)])
            return 0
        lax.fori_loop(0, ept // EC, _zero_sh, 0)
        plsc.subcore_barrier()
        _p0.__exit__(None, None, None)
        _p1 = jax.named_scope("p1_accum"); _p1.__enter__()

        def _grp1(q, _):
            r0 = s * nch + q * SG
            pltpu.sync_copy(gidx_hbm.at[c, pl.ds(r0, SG)], gidx_v)
            pltpu.sync_copy(eid_hbm.at[pl.ds(r0, SG)], eid_v)
            pltpu.async_copy(table_hbm.at[gidx_v.at[0]], rows_v.at[0], sem_g)
            pltpu.async_copy(table_hbm.at[gidx_v.at[1]], rows_v.at[1], sem_g)
            pltpu.async_copy(table_hbm.at[gidx_v.at[2]], rows_v.at[2], sem_g)

            def _chunk(j, _):
                cur = lax.rem(j, 4)
                pltpu.make_async_copy(
                    table_hbm.at[gidx_v.at[j]], rows_v.at[cur], sem_g).wait()

                @pl.when(j >= 1)
                def _():
                    pltpu.make_async_copy(
                        rows_v.at[lax.rem(j + 3, 4)],
                        agg_sh.at[eid_v.at[j - 1]], sem_s).wait()

                @pl.when(j + 3 < SG)
                def _():
                    pltpu.async_copy(
                        table_hbm.at[gidx_v.at[j + 3]],
                        rows_v.at[lax.rem(j + 3, 4)], sem_g)

                pltpu.async_copy(
                    rows_v.at[cur], agg_sh.at[eid_v.at[j]], sem_s, add=True)
                pltpu.async_copy(ones_v, deg_sh.at[eid_v.at[j]], sem_d,
                                 add=True)
                return 0
            lax.fori_loop(0, SG, _chunk, 0)
            pltpu.make_async_copy(
                rows_v.at[(SG - 1) % 4], agg_sh.at[eid_v.at[SG - 1]],
                sem_s).wait()

            def _drain(j, _):
                pltpu.make_async_copy(
                    ones_v, deg_sh.at[eid_v.at[0]], sem_d).wait()
                return 0
            lax.fori_loop(0, SG, _drain, 0)
            return 0
        lax.fori_loop(0, nch // SG, _grp1, 0)
        plsc.subcore_barrier()
        _p1.__exit__(None, None, None)
        _p2 = jax.named_scope("p2_scale"); _p2.__enter__()

        ng2 = ept // EC

        def _stage2(q, buf):
            eq = e0 + q * EC
            pltpu.async_copy(agg_sh.at[pl.ds(eq, EC)], rows_v.at[buf].at[pl.ds(0, EC)], sem_g)
            pltpu.async_copy(deg_sh.at[pl.ds(eq, EC)], dstg_v.at[pl.ds(buf * EC, EC)], sem_g)

        def _unstage2(q, buf):
            eq = e0 + q * EC
            pltpu.make_async_copy(agg_sh.at[pl.ds(eq, EC)], rows_v.at[buf].at[pl.ds(0, EC)], sem_g).wait()
            pltpu.make_async_copy(deg_sh.at[pl.ds(eq, EC)], dstg_v.at[pl.ds(buf * EC, EC)], sem_g).wait()

        def _wrwait2(q, buf):
            eq = e0 + q * EC
            pltpu.make_async_copy(rows_v.at[buf].at[pl.ds(0, EC)], agg_sh.at[pl.ds(eq, EC)], sem_s).wait()
            pltpu.make_async_copy(rows_v.at[buf].at[pl.ds(0, EC)], out2_hbm.at[pl.ds(eq, EC), pl.ds(c * hc, hc)], sem_w).wait()

        _stage2(0, 0)

        def _grp2(q, _):
            eq = e0 + q * EC
            cur = lax.rem(q, 2)
            nxt = 1 - cur
            _unstage2(q, cur)

            @pl.when(q >= 1)
            def _():
                _wrwait2(q - 1, nxt)

            @pl.when(q + 1 < ng2)
            def _():
                _stage2(q + 1, nxt)

            def _scale_row(i, _):
                ridx = jnp.full((L,), cur * EC + i, jnp.int32)
                cidx = jnp.zeros((L,), jnp.int32)
                inv = 1.0 / plsc.load_gather(dstg_v, [ridx, cidx])
                for k in range(hc // L):
                    sl = pl.ds(k * L, L)
                    rows_v[cur, i, sl] = rows_v[cur, i, sl] * inv
                return 0
            lax.fori_loop(0, EC, _scale_row, 0)
            pltpu.async_copy(rows_v.at[cur].at[pl.ds(0, EC)], agg_sh.at[pl.ds(eq, EC)], sem_s)
            pltpu.async_copy(rows_v.at[cur].at[pl.ds(0, EC)], out2_hbm.at[pl.ds(eq, EC), pl.ds(c * hc, hc)], sem_w)
            return 0
        lax.fori_loop(0, ng2, _grp2, 0)
        _wrwait2(ng2 - 1, lax.rem(ng2 - 1, 2))
        plsc.subcore_barrier()
        _p2.__exit__(None, None, None)
        _p3 = jax.named_scope("p3_back"); _p3.__enter__()

        def _grp3(q, _):
            pltpu.sync_copy(eid_hbm.at[pl.ds(s * nch + q * SG, SG)], eid_v)
            pltpu.async_copy(agg_sh.at[eid_v.at[0]], rows_v.at[0], sem_g)
            pltpu.async_copy(agg_sh.at[eid_v.at[1]], rows_v.at[1], sem_g)
            pltpu.async_copy(agg_sh.at[eid_v.at[2]], rows_v.at[2], sem_g)

            def _chunk(j, _):
                cur = lax.rem(j, 4)
                pltpu.make_async_copy(
                    agg_sh.at[eid_v.at[j]], rows_v.at[cur], sem_g).wait()

                @pl.when(j >= 1)
                def _():
                    pltpu.make_async_copy(
                        rows_v.at[lax.rem(j + 3, 4)],
                        out1_hbm.at[pl.ds(a0 + (q * SG + j - 1) * B, B),
                                    pl.ds(c * hc, hc)],
                        sem_w).wait()

                @pl.when(j + 3 < SG)
                def _():
                    pltpu.async_copy(
                        agg_sh.at[eid_v.at[j + 3]],
                        rows_v.at[lax.rem(j + 3, 4)], sem_g)

                pltpu.async_copy(
                    rows_v.at[cur],
                    out1_hbm.at[pl.ds(a0 + (q * SG + j) * B, B),
                                pl.ds(c * hc, hc)], sem_w)
                return 0
            lax.fori_loop(0, SG, _chunk, 0)
            pltpu.make_async_copy(
                rows_v.at[(SG - 1) % 4],
                out1_hbm.at[pl.ds(a0 + (q * SG + SG - 1) * B, B),
                            pl.ds(c * hc, hc)],
                sem_w).wait()
            return 0
        lax.fori_loop(0, nch // SG, _grp3, 0)
        _p3.__exit__(None, None, None)

    return sc_fn


def kernel(nodes_representations, hyperedge_arg_node_idxs,
           unq_hyperedge_type_reprs, hyperedge_type_name_unq_idxs,
           unq_hyperedge_arg_name_reprs, hyperedge_arg_name_unq_idxs,
           hyperedge_arg_to_edge_id, num_edges, W):
    n, h = nodes_representations.shape
    a = hyperedge_arg_node_idxs.shape[0]
    e = hyperedge_type_name_unq_idxs.shape[0]

    table = _matmul_split(nodes_representations, W).reshape(NC * n, h // NC)

    idx = hyperedge_arg_node_idxs.astype(jnp.int32)
    gidx = jnp.stack([idx, idx + n]).reshape(NC, a // B, B)
    eid2 = hyperedge_arg_to_edge_id.astype(jnp.int32).reshape(a // B, B)

    ones8 = jnp.ones((B, 8), jnp.float32)
    out1, out2 = _make_sc(n, h, a, e)(table, gidx, eid2, ones8)
    return out1, out2


# speedup vs baseline: 1.0555x; 1.0437x over previous
import functools

import jax
import jax.numpy as jnp
from jax import lax
from jax.experimental import pallas as pl
from jax.experimental.pallas import tpu as pltpu
from jax.experimental.pallas import tpu_sc as plsc

NC = 2
NS = 16
L = 16


def _mm_body(x_ref, w_ref, o_ref):
    o_ref[...] = lax.dot_general(x_ref[...], w_ref[...],
                                 (((1,), (1,)), ((), ())),
                                 preferred_element_type=jnp.float32)


def _matmul_split(nodes, w):
    n, h = nodes.shape
    bn = 2000
    return pl.pallas_call(
        _mm_body,
        grid=(n // bn,),
        in_specs=[
            pl.BlockSpec((bn, h), lambda i: (i, 0)),
            pl.BlockSpec((h, h), lambda i: (0, 0)),
        ],
        out_specs=pl.BlockSpec((bn, h), lambda i: (i, 0)),
        out_shape=jax.ShapeDtypeStruct((n, h), jnp.float32),
    )(nodes, w)


B = 125
SG = 10
EC = 50


@functools.cache
def _make_sc(n, h, a, e):
    hc = h // NC
    pw = a // NS
    nch = pw // B
    ept = e // NS
    mesh = plsc.VectorSubcoreMesh(core_axis_name="c", subcore_axis_name="s")

    @functools.partial(
        pl.kernel,
        out_type=[
            jax.ShapeDtypeStruct((a, h), jnp.float32),
            jax.ShapeDtypeStruct((e, h), jnp.float32),
        ],
        mesh=mesh,
        compiler_params=pltpu.CompilerParams(use_tc_tiling_on_sc=False, needs_layout_passes=False),
        scratch_types=[
            pltpu.VMEM((SG, B), jnp.int32),
            pltpu.VMEM((SG, B), jnp.int32),
            pltpu.VMEM((4, B, hc), jnp.float32),
            pltpu.VMEM((B, 8), jnp.float32),
            pltpu.VMEM((2 * EC, 8), jnp.float32),
            pltpu.VMEM_SHARED((e, hc), jnp.float32),
            pltpu.VMEM_SHARED((e, 8), jnp.float32),
            pltpu.SemaphoreType.DMA,
            pltpu.SemaphoreType.DMA,
            pltpu.SemaphoreType.DMA,
            pltpu.SemaphoreType.DMA,
        ],
    )
    def sc_fn(table_hbm, gidx_hbm, eid_hbm, ones_hbm, out1_hbm, out2_hbm,
              gidx_v, eid_v, rows_v, ones_v, dstg_v,
              agg_sh, deg_sh, sem_g, sem_s, sem_w, sem_d):
        c = lax.axis_index("c")
        s = lax.axis_index("s")
        e0 = s * ept
        a0 = s * pw

        _p0 = jax.named_scope("p0_zero"); _p0.__enter__()
        pltpu.sync_copy(ones_hbm, ones_v)

        def _zero_row(i, _):
            z = jnp.zeros((L,), jnp.float32)
            for k in range(hc // L):
                rows_v[0, i, pl.ds(k * L, L)] = z
            return 0
        lax.fori_loop(0, B, _zero_row, 0)

        def _zero_sh(q, _):
            pltpu.sync_copy(rows_v.at[0],
                            agg_sh.at[pl.ds(e0 + q * B, B)])
            pltpu.sync_copy(rows_v.at[0].at[pl.ds(0, B), pl.ds(0, 8)],
                            deg_sh.at[pl.ds(e0 + q * B, B)])
            return 0
        lax.fori_loop(0, ept // B, _zero_sh, 0)
        plsc.subcore_barrier()
        _p0.__exit__(None, None, None)
        _p1 = jax.named_scope("p1_accum"); _p1.__enter__()

        def _grp1(q, _):
            r0 = s * nch + q * SG
            pltpu.sync_copy(gidx_hbm.at[c, pl.ds(r0, SG)], gidx_v)
            pltpu.sync_copy(eid_hbm.at[pl.ds(r0, SG)], eid_v)
            pltpu.async_copy(table_hbm.at[gidx_v.at[0]], rows_v.at[0], sem_g)
            pltpu.async_copy(table_hbm.at[gidx_v.at[1]], rows_v.at[1], sem_g)
            pltpu.async_copy(table_hbm.at[gidx_v.at[2]], rows_v.at[2], sem_g)

            def _chunk(j, _):
                cur = lax.rem(j, 4)
                pltpu.make_async_copy(
                    table_hbm.at[gidx_v.at[j]], rows_v.at[cur], sem_g).wait()

                @pl.when(j >= 1)
                def _():
                    pltpu.make_async_copy(
                        rows_v.at[lax.rem(j + 3, 4)],
                        agg_sh.at[eid_v.at[j - 1]], sem_s).wait()

                @pl.when(j + 3 < SG)
                def _():
                    pltpu.async_copy(
                        table_hbm.at[gidx_v.at[j + 3]],
                        rows_v.at[lax.rem(j + 3, 4)], sem_g)

                pltpu.async_copy(
                    rows_v.at[cur], agg_sh.at[eid_v.at[j]], sem_s, add=True)
                pltpu.async_copy(ones_v, deg_sh.at[eid_v.at[j]], sem_d,
                                 add=True)
                return 0
            lax.fori_loop(0, SG, _chunk, 0)
            pltpu.make_async_copy(
                rows_v.at[(SG - 1) % 4], agg_sh.at[eid_v.at[SG - 1]],
                sem_s).wait()

            def _drain(j, _):
                pltpu.make_async_copy(
                    ones_v, deg_sh.at[eid_v.at[0]], sem_d).wait()
                return 0
            lax.fori_loop(0, SG, _drain, 0)
            return 0
        lax.fori_loop(0, nch // SG, _grp1, 0)
        plsc.subcore_barrier()
        _p1.__exit__(None, None, None)
        _p2 = jax.named_scope("p2_scale"); _p2.__enter__()

        ng2 = ept // EC

        def _stage2(q, buf):
            eq = e0 + q * EC
            pltpu.async_copy(agg_sh.at[pl.ds(eq, EC)], rows_v.at[buf].at[pl.ds(0, EC)], sem_g)
            pltpu.async_copy(deg_sh.at[pl.ds(eq, EC)], dstg_v.at[pl.ds(buf * EC, EC)], sem_g)

        def _unstage2(q, buf):
            eq = e0 + q * EC
            pltpu.make_async_copy(agg_sh.at[pl.ds(eq, EC)], rows_v.at[buf].at[pl.ds(0, EC)], sem_g).wait()
            pltpu.make_async_copy(deg_sh.at[pl.ds(eq, EC)], dstg_v.at[pl.ds(buf * EC, EC)], sem_g).wait()

        def _wrwait2(q, buf):
            eq = e0 + q * EC
            pltpu.make_async_copy(rows_v.at[buf].at[pl.ds(0, EC)], agg_sh.at[pl.ds(eq, EC)], sem_s).wait()
            pltpu.make_async_copy(rows_v.at[buf].at[pl.ds(0, EC)], out2_hbm.at[pl.ds(eq, EC), pl.ds(c * hc, hc)], sem_w).wait()

        _stage2(0, 0)

        def _grp2(q, _):
            eq = e0 + q * EC
            cur = lax.rem(q, 2)
            nxt = 1 - cur
            _unstage2(q, cur)

            @pl.when(q >= 1)
            def _():
                _wrwait2(q - 1, nxt)

            @pl.when(q + 1 < ng2)
            def _():
                _stage2(q + 1, nxt)

            def _scale_row(i, _):
                ridx = jnp.full((L,), cur * EC + i, jnp.int32)
                cidx = jnp.zeros((L,), jnp.int32)
                inv = 1.0 / plsc.load_gather(dstg_v, [ridx, cidx])
                for k in range(hc // L):
                    sl = pl.ds(k * L, L)
                    rows_v[cur, i, sl] = rows_v[cur, i, sl] * inv
                return 0
            lax.fori_loop(0, EC, _scale_row, 0)
            pltpu.async_copy(rows_v.at[cur].at[pl.ds(0, EC)], agg_sh.at[pl.ds(eq, EC)], sem_s)
            pltpu.async_copy(rows_v.at[cur].at[pl.ds(0, EC)], out2_hbm.at[pl.ds(eq, EC), pl.ds(c * hc, hc)], sem_w)
            return 0
        lax.fori_loop(0, ng2, _grp2, 0)
        _wrwait2(ng2 - 1, lax.rem(ng2 - 1, 2))
        plsc.subcore_barrier()
        _p2.__exit__(None, None, None)
        _p3 = jax.named_scope("p3_back"); _p3.__enter__()

        def _grp3(q, _):
            pltpu.sync_copy(eid_hbm.at[pl.ds(s * nch + q * SG, SG)], eid_v)
            pltpu.async_copy(agg_sh.at[eid_v.at[0]], rows_v.at[0], sem_g)
            pltpu.async_copy(agg_sh.at[eid_v.at[1]], rows_v.at[1], sem_g)
            pltpu.async_copy(agg_sh.at[eid_v.at[2]], rows_v.at[2], sem_g)

            def _chunk(j, _):
                cur = lax.rem(j, 4)
                pltpu.make_async_copy(
                    agg_sh.at[eid_v.at[j]], rows_v.at[cur], sem_g).wait()

                @pl.when(j >= 1)
                def _():
                    pltpu.make_async_copy(
                        rows_v.at[lax.rem(j + 3, 4)],
                        out1_hbm.at[pl.ds(a0 + (q * SG + j - 1) * B, B),
                                    pl.ds(c * hc, hc)],
                        sem_w).wait()

                @pl.when(j + 3 < SG)
                def _():
                    pltpu.async_copy(
                        agg_sh.at[eid_v.at[j + 3]],
                        rows_v.at[lax.rem(j + 3, 4)], sem_g)

                pltpu.async_copy(
                    rows_v.at[cur],
                    out1_hbm.at[pl.ds(a0 + (q * SG + j) * B, B),
                                pl.ds(c * hc, hc)], sem_w)
                return 0
            lax.fori_loop(0, SG, _chunk, 0)
            pltpu.make_async_copy(
                rows_v.at[(SG - 1) % 4],
                out1_hbm.at[pl.ds(a0 + (q * SG + SG - 1) * B, B),
                            pl.ds(c * hc, hc)],
                sem_w).wait()
            return 0
        lax.fori_loop(0, nch // SG, _grp3, 0)
        _p3.__exit__(None, None, None)

    return sc_fn


def kernel(nodes_representations, hyperedge_arg_node_idxs,
           unq_hyperedge_type_reprs, hyperedge_type_name_unq_idxs,
           unq_hyperedge_arg_name_reprs, hyperedge_arg_name_unq_idxs,
           hyperedge_arg_to_edge_id, num_edges, W):
    n, h = nodes_representations.shape
    a = hyperedge_arg_node_idxs.shape[0]
    e = hyperedge_type_name_unq_idxs.shape[0]

    table = _matmul_split(nodes_representations, W).reshape(NC * n, h // NC)

    idx2 = 2 * hyperedge_arg_node_idxs.astype(jnp.int32)
    gidx = jnp.stack([idx2, idx2 + 1]).reshape(NC, a // B, B)
    eid2 = hyperedge_arg_to_edge_id.astype(jnp.int32).reshape(a // B, B)

    ones8 = jnp.ones((B, 8), jnp.float32)
    out1, out2 = _make_sc(n, h, a, e)(table, gidx, eid2, ones8)
    return out1, out2


# speedup vs baseline: 1.1289x; 1.0696x over previous
import functools

import jax
import jax.numpy as jnp
from jax import lax
from jax.experimental import pallas as pl
from jax.experimental.pallas import tpu as pltpu
from jax.experimental.pallas import tpu_sc as plsc

NC = 2
NS = 16
L = 16


def _mm_body(x_ref, w_ref, o_ref):
    o_ref[...] = lax.dot_general(x_ref[...], w_ref[...],
                                 (((1,), (1,)), ((), ())),
                                 preferred_element_type=jnp.float32)


def _matmul_split(nodes, w):
    n, h = nodes.shape
    bn = 2000
    return pl.pallas_call(
        _mm_body,
        grid=(n // bn,),
        in_specs=[
            pl.BlockSpec((bn, h), lambda i: (i, 0)),
            pl.BlockSpec((h, h), lambda i: (0, 0)),
        ],
        out_specs=pl.BlockSpec((bn, h), lambda i: (i, 0)),
        out_shape=jax.ShapeDtypeStruct((n, h), jnp.float32),
    )(nodes, w)


B = 125
SG = 20
EC = 50


@functools.cache
def _make_sc(n, h, a, e):
    hc = h // NC
    pw = a // NS
    nch = pw // B
    ept = e // NS
    mesh = plsc.VectorSubcoreMesh(core_axis_name="c", subcore_axis_name="s")

    @functools.partial(
        pl.kernel,
        out_type=[
            jax.ShapeDtypeStruct((a, h), jnp.float32),
            jax.ShapeDtypeStruct((e, h), jnp.float32),
        ],
        mesh=mesh,
        compiler_params=pltpu.CompilerParams(use_tc_tiling_on_sc=False, needs_layout_passes=False),
        scratch_types=[
            pltpu.VMEM((SG, B), jnp.int32),
            pltpu.VMEM((SG, B), jnp.int32),
            pltpu.VMEM((4, B, hc), jnp.float32),
            pltpu.VMEM((B, 8), jnp.float32),
            pltpu.VMEM((2 * EC, 8), jnp.float32),
            pltpu.VMEM_SHARED((e, hc), jnp.float32),
            pltpu.VMEM_SHARED((e, 8), jnp.float32),
            pltpu.SemaphoreType.DMA,
            pltpu.SemaphoreType.DMA,
            pltpu.SemaphoreType.DMA,
            pltpu.SemaphoreType.DMA,
        ],
    )
    def sc_fn(table_hbm, gidx_hbm, eid_hbm, ones_hbm, out1_hbm, out2_hbm,
              gidx_v, eid_v, rows_v, ones_v, dstg_v,
              agg_sh, deg_sh, sem_g, sem_s, sem_w, sem_d):
        c = lax.axis_index("c")
        s = lax.axis_index("s")
        e0 = s * ept
        a0 = s * pw

        _p0 = jax.named_scope("p0_zero"); _p0.__enter__()
        pltpu.sync_copy(ones_hbm, ones_v)

        def _zero_row(i, _):
            z = jnp.zeros((L,), jnp.float32)
            for k in range(hc // L):
                rows_v[0, i, pl.ds(k * L, L)] = z
            return 0
        lax.fori_loop(0, B, _zero_row, 0)

        def _zero_sh(q, _):
            pltpu.sync_copy(rows_v.at[0],
                            agg_sh.at[pl.ds(e0 + q * B, B)])
            pltpu.sync_copy(rows_v.at[0].at[pl.ds(0, B), pl.ds(0, 8)],
                            deg_sh.at[pl.ds(e0 + q * B, B)])
            return 0
        lax.fori_loop(0, ept // B, _zero_sh, 0)
        plsc.subcore_barrier()
        _p0.__exit__(None, None, None)
        _p1 = jax.named_scope("p1_accum"); _p1.__enter__()

        def _grp1(q, _):
            r0 = s * nch + q * SG
            pltpu.sync_copy(gidx_hbm.at[c, pl.ds(r0, SG)], gidx_v)
            pltpu.sync_copy(eid_hbm.at[pl.ds(r0, SG)], eid_v)
            pltpu.async_copy(table_hbm.at[gidx_v.at[0]], rows_v.at[0], sem_g)
            pltpu.async_copy(table_hbm.at[gidx_v.at[1]], rows_v.at[1], sem_g)
            pltpu.async_copy(table_hbm.at[gidx_v.at[2]], rows_v.at[2], sem_g)

            def _chunk(j, _):
                cur = lax.rem(j, 4)
                pltpu.make_async_copy(
                    table_hbm.at[gidx_v.at[j]], rows_v.at[cur], sem_g).wait()

                @pl.when(j >= 1)
                def _():
                    pltpu.make_async_copy(
                        rows_v.at[lax.rem(j + 3, 4)],
                        agg_sh.at[eid_v.at[j - 1]], sem_s).wait()

                @pl.when(j + 3 < SG)
                def _():
                    pltpu.async_copy(
                        table_hbm.at[gidx_v.at[j + 3]],
                        rows_v.at[lax.rem(j + 3, 4)], sem_g)

                pltpu.async_copy(
                    rows_v.at[cur], agg_sh.at[eid_v.at[j]], sem_s, add=True)
                pltpu.async_copy(ones_v, deg_sh.at[eid_v.at[j]], sem_d,
                                 add=True)
                return 0
            lax.fori_loop(0, SG, _chunk, 0)
            pltpu.make_async_copy(
                rows_v.at[(SG - 1) % 4], agg_sh.at[eid_v.at[SG - 1]],
                sem_s).wait()

            def _drain(j, _):
                pltpu.make_async_copy(
                    ones_v, deg_sh.at[eid_v.at[0]], sem_d).wait()
                return 0
            lax.fori_loop(0, SG, _drain, 0)
            return 0
        lax.fori_loop(0, nch // SG, _grp1, 0)
        plsc.subcore_barrier()
        _p1.__exit__(None, None, None)
        _p2 = jax.named_scope("p2_scale"); _p2.__enter__()

        ng2 = ept // EC

        def _stage2(q, buf):
            eq = e0 + q * EC
            pltpu.async_copy(agg_sh.at[pl.ds(eq, EC)], rows_v.at[buf].at[pl.ds(0, EC)], sem_g)
            pltpu.async_copy(deg_sh.at[pl.ds(eq, EC)], dstg_v.at[pl.ds(buf * EC, EC)], sem_g)

        def _unstage2(q, buf):
            eq = e0 + q * EC
            pltpu.make_async_copy(agg_sh.at[pl.ds(eq, EC)], rows_v.at[buf].at[pl.ds(0, EC)], sem_g).wait()
            pltpu.make_async_copy(deg_sh.at[pl.ds(eq, EC)], dstg_v.at[pl.ds(buf * EC, EC)], sem_g).wait()

        def _wrwait2(q, buf):
            eq = e0 + q * EC
            pltpu.make_async_copy(rows_v.at[buf].at[pl.ds(0, EC)], agg_sh.at[pl.ds(eq, EC)], sem_s).wait()
            pltpu.make_async_copy(rows_v.at[buf].at[pl.ds(0, EC)], out2_hbm.at[pl.ds(eq, EC), pl.ds(c * hc, hc)], sem_w).wait()

        _stage2(0, 0)

        def _grp2(q, _):
            eq = e0 + q * EC
            cur = lax.rem(q, 2)
            nxt = 1 - cur
            _unstage2(q, cur)

            @pl.when(q >= 1)
            def _():
                _wrwait2(q - 1, nxt)

            @pl.when(q + 1 < ng2)
            def _():
                _stage2(q + 1, nxt)

            def _scale_row(i, _):
                ridx = jnp.full((L,), cur * EC + i, jnp.int32)
                cidx = jnp.zeros((L,), jnp.int32)
                inv = 1.0 / plsc.load_gather(dstg_v, [ridx, cidx])
                for k in range(hc // L):
                    sl = pl.ds(k * L, L)
                    rows_v[cur, i, sl] = rows_v[cur, i, sl] * inv
                return 0
            lax.fori_loop(0, EC, _scale_row, 0)
            pltpu.async_copy(rows_v.at[cur].at[pl.ds(0, EC)], agg_sh.at[pl.ds(eq, EC)], sem_s)
            pltpu.async_copy(rows_v.at[cur].at[pl.ds(0, EC)], out2_hbm.at[pl.ds(eq, EC), pl.ds(c * hc, hc)], sem_w)
            return 0
        lax.fori_loop(0, ng2, _grp2, 0)
        _wrwait2(ng2 - 1, lax.rem(ng2 - 1, 2))
        plsc.subcore_barrier()
        _p2.__exit__(None, None, None)
        _p3 = jax.named_scope("p3_back"); _p3.__enter__()

        def _grp3(q, _):
            pltpu.sync_copy(eid_hbm.at[pl.ds(s * nch + q * SG, SG)], eid_v)
            pltpu.async_copy(agg_sh.at[eid_v.at[0]], rows_v.at[0], sem_g)
            pltpu.async_copy(agg_sh.at[eid_v.at[1]], rows_v.at[1], sem_g)
            pltpu.async_copy(agg_sh.at[eid_v.at[2]], rows_v.at[2], sem_g)

            def _chunk(j, _):
                cur = lax.rem(j, 4)
                pltpu.make_async_copy(
                    agg_sh.at[eid_v.at[j]], rows_v.at[cur], sem_g).wait()

                @pl.when(j >= 1)
                def _():
                    pltpu.make_async_copy(
                        rows_v.at[lax.rem(j + 3, 4)],
                        out1_hbm.at[pl.ds(a0 + (q * SG + j - 1) * B, B),
                                    pl.ds(c * hc, hc)],
                        sem_w).wait()

                @pl.when(j + 3 < SG)
                def _():
                    pltpu.async_copy(
                        agg_sh.at[eid_v.at[j + 3]],
                        rows_v.at[lax.rem(j + 3, 4)], sem_g)

                pltpu.async_copy(
                    rows_v.at[cur],
                    out1_hbm.at[pl.ds(a0 + (q * SG + j) * B, B),
                                pl.ds(c * hc, hc)], sem_w)
                return 0
            lax.fori_loop(0, SG, _chunk, 0)
            pltpu.make_async_copy(
                rows_v.at[(SG - 1) % 4],
                out1_hbm.at[pl.ds(a0 + (q * SG + SG - 1) * B, B),
                            pl.ds(c * hc, hc)],
                sem_w).wait()
            return 0
        lax.fori_loop(0, nch // SG, _grp3, 0)
        _p3.__exit__(None, None, None)

    return sc_fn


def kernel(nodes_representations, hyperedge_arg_node_idxs,
           unq_hyperedge_type_reprs, hyperedge_type_name_unq_idxs,
           unq_hyperedge_arg_name_reprs, hyperedge_arg_name_unq_idxs,
           hyperedge_arg_to_edge_id, num_edges, W):
    n, h = nodes_representations.shape
    a = hyperedge_arg_node_idxs.shape[0]
    e = hyperedge_type_name_unq_idxs.shape[0]

    table = _matmul_split(nodes_representations, W).reshape(NC * n, h // NC)

    idx2 = 2 * hyperedge_arg_node_idxs.astype(jnp.int32)
    gidx = jnp.stack([idx2, idx2 + 1]).reshape(NC, a // B, B)
    eid2 = hyperedge_arg_to_edge_id.astype(jnp.int32).reshape(a // B, B)

    ones8 = jnp.ones((B, 8), jnp.float32)
    out1, out2 = _make_sc(n, h, a, e)(table, gidx, eid2, ones8)
    return out1, out2


# speedup vs baseline: 1.1517x; 1.0202x over previous
import functools

import jax
import jax.numpy as jnp
from jax import lax
from jax.experimental import pallas as pl
from jax.experimental.pallas import tpu as pltpu
from jax.experimental.pallas import tpu_sc as plsc

NC = 2
NS = 16
L = 16


def _mm_body(x_ref, w_ref, o_ref):
    o_ref[...] = lax.dot_general(x_ref[...], w_ref[...],
                                 (((1,), (1,)), ((), ())),
                                 preferred_element_type=jnp.float32)


def _matmul_split(nodes, w):
    n, h = nodes.shape
    bn = 2000
    return pl.pallas_call(
        _mm_body,
        grid=(n // bn,),
        in_specs=[
            pl.BlockSpec((bn, h), lambda i: (i, 0)),
            pl.BlockSpec((h, h), lambda i: (0, 0)),
        ],
        out_specs=pl.BlockSpec((bn, h), lambda i: (i, 0)),
        out_shape=jax.ShapeDtypeStruct((n, h), jnp.float32),
    )(nodes, w)


B = 125
SG = 32
EC = 50


@functools.cache
def _make_sc(n, h, a, e):
    hc = h // NC
    pw = a // NS
    nch = pw // B
    ept = e // NS
    mesh = plsc.VectorSubcoreMesh(core_axis_name="c", subcore_axis_name="s")

    @functools.partial(
        pl.kernel,
        out_type=[
            jax.ShapeDtypeStruct((a, h), jnp.float32),
            jax.ShapeDtypeStruct((e, h), jnp.float32),
        ],
        mesh=mesh,
        compiler_params=pltpu.CompilerParams(use_tc_tiling_on_sc=False, needs_layout_passes=False),
        scratch_types=[
            pltpu.VMEM((SG, B), jnp.int32),
            pltpu.VMEM((SG, B), jnp.int32),
            pltpu.VMEM((3, B, hc), jnp.float32),
            pltpu.VMEM((B, 8), jnp.float32),
            pltpu.VMEM((2 * EC, 8), jnp.float32),
            pltpu.VMEM_SHARED((e, hc), jnp.float32),
            pltpu.VMEM_SHARED((e, 8), jnp.float32),
            pltpu.SemaphoreType.DMA,
            pltpu.SemaphoreType.DMA,
            pltpu.SemaphoreType.DMA,
            pltpu.SemaphoreType.DMA,
        ],
    )
    def sc_fn(table_hbm, gidx_hbm, eid_hbm, ones_hbm, out1_hbm, out2_hbm,
              gidx_v, eid_v, rows_v, ones_v, dstg_v,
              agg_sh, deg_sh, sem_g, sem_s, sem_w, sem_d):
        c = lax.axis_index("c")
        s = lax.axis_index("s")
        e0 = s * ept
        a0 = s * pw

        _p0 = jax.named_scope("p0_zero"); _p0.__enter__()
        pltpu.sync_copy(ones_hbm, ones_v)

        def _zero_row(i, _):
            z = jnp.zeros((L,), jnp.float32)
            for k in range(hc // L):
                rows_v[0, i, pl.ds(k * L, L)] = z
            return 0
        lax.fori_loop(0, B, _zero_row, 0)

        def _zero_sh(q, _):
            pltpu.sync_copy(rows_v.at[0],
                            agg_sh.at[pl.ds(e0 + q * B, B)])
            pltpu.sync_copy(rows_v.at[0].at[pl.ds(0, B), pl.ds(0, 8)],
                            deg_sh.at[pl.ds(e0 + q * B, B)])
            return 0
        lax.fori_loop(0, ept // B, _zero_sh, 0)
        plsc.subcore_barrier()
        _p0.__exit__(None, None, None)
        _p1 = jax.named_scope("p1_accum"); _p1.__enter__()

        def _grp1(q, _):
            r0 = s * nch + q * SG
            pltpu.sync_copy(gidx_hbm.at[c, pl.ds(r0, SG)], gidx_v)
            pltpu.sync_copy(eid_hbm.at[pl.ds(r0, SG)], eid_v)
            pltpu.async_copy(table_hbm.at[gidx_v.at[0]], rows_v.at[0], sem_g)
            pltpu.async_copy(table_hbm.at[gidx_v.at[1]], rows_v.at[1], sem_g)

            def _chunk(j, _):
                cur = lax.rem(j, 3)
                pltpu.make_async_copy(
                    table_hbm.at[gidx_v.at[j]], rows_v.at[cur], sem_g).wait()

                @pl.when(j >= 1)
                def _():
                    pltpu.make_async_copy(
                        rows_v.at[lax.rem(j + 2, 3)],
                        agg_sh.at[eid_v.at[j - 1]], sem_s).wait()

                @pl.when(j + 2 < SG)
                def _():
                    pltpu.async_copy(
                        table_hbm.at[gidx_v.at[j + 2]],
                        rows_v.at[lax.rem(j + 2, 3)], sem_g)

                pltpu.async_copy(
                    rows_v.at[cur], agg_sh.at[eid_v.at[j]], sem_s, add=True)
                pltpu.async_copy(ones_v, deg_sh.at[eid_v.at[j]], sem_d,
                                 add=True)
                return 0
            lax.fori_loop(0, SG, _chunk, 0)
            pltpu.make_async_copy(
                rows_v.at[(SG - 1) % 3], agg_sh.at[eid_v.at[SG - 1]],
                sem_s).wait()

            def _drain(j, _):
                pltpu.make_async_copy(
                    ones_v, deg_sh.at[eid_v.at[0]], sem_d).wait()
                return 0
            lax.fori_loop(0, SG, _drain, 0)
            return 0
        lax.fori_loop(0, nch // SG, _grp1, 0)
        plsc.subcore_barrier()
        _p1.__exit__(None, None, None)
        _p2 = jax.named_scope("p2_scale"); _p2.__enter__()

        ng2 = ept // EC

        def _stage2(q, buf):
            eq = e0 + q * EC
            pltpu.async_copy(agg_sh.at[pl.ds(eq, EC)], rows_v.at[buf].at[pl.ds(0, EC)], sem_g)
            pltpu.async_copy(deg_sh.at[pl.ds(eq, EC)], dstg_v.at[pl.ds(buf * EC, EC)], sem_g)

        def _unstage2(q, buf):
            eq = e0 + q * EC
            pltpu.make_async_copy(agg_sh.at[pl.ds(eq, EC)], rows_v.at[buf].at[pl.ds(0, EC)], sem_g).wait()
            pltpu.make_async_copy(deg_sh.at[pl.ds(eq, EC)], dstg_v.at[pl.ds(buf * EC, EC)], sem_g).wait()

        def _wrwait2(q, buf):
            eq = e0 + q * EC
            pltpu.make_async_copy(rows_v.at[buf].at[pl.ds(0, EC)], agg_sh.at[pl.ds(eq, EC)], sem_s).wait()
            pltpu.make_async_copy(rows_v.at[buf].at[pl.ds(0, EC)], out2_hbm.at[pl.ds(eq, EC), pl.ds(c * hc, hc)], sem_w).wait()

        _stage2(0, 0)

        def _grp2(q, _):
            eq = e0 + q * EC
            cur = lax.rem(q, 2)
            nxt = 1 - cur
            _unstage2(q, cur)

            @pl.when(q >= 1)
            def _():
                _wrwait2(q - 1, nxt)

            @pl.when(q + 1 < ng2)
            def _():
                _stage2(q + 1, nxt)

            def _scale_row(i, _):
                ridx = jnp.full((L,), cur * EC + i, jnp.int32)
                cidx = jnp.zeros((L,), jnp.int32)
                inv = 1.0 / plsc.load_gather(dstg_v, [ridx, cidx])
                for k in range(hc // L):
                    sl = pl.ds(k * L, L)
                    rows_v[cur, i, sl] = rows_v[cur, i, sl] * inv
                return 0
            lax.fori_loop(0, EC, _scale_row, 0)
            pltpu.async_copy(rows_v.at[cur].at[pl.ds(0, EC)], agg_sh.at[pl.ds(eq, EC)], sem_s)
            pltpu.async_copy(rows_v.at[cur].at[pl.ds(0, EC)], out2_hbm.at[pl.ds(eq, EC), pl.ds(c * hc, hc)], sem_w)
            return 0
        lax.fori_loop(0, ng2, _grp2, 0)
        _wrwait2(ng2 - 1, lax.rem(ng2 - 1, 2))
        plsc.subcore_barrier()
        _p2.__exit__(None, None, None)
        _p3 = jax.named_scope("p3_back"); _p3.__enter__()

        def _grp3(q, _):
            pltpu.sync_copy(eid_hbm.at[pl.ds(s * nch + q * SG, SG)], eid_v)
            pltpu.async_copy(agg_sh.at[eid_v.at[0]], rows_v.at[0], sem_g)
            pltpu.async_copy(agg_sh.at[eid_v.at[1]], rows_v.at[1], sem_g)

            def _chunk(j, _):
                cur = lax.rem(j, 3)
                pltpu.make_async_copy(
                    agg_sh.at[eid_v.at[j]], rows_v.at[cur], sem_g).wait()

                @pl.when(j >= 1)
                def _():
                    pltpu.make_async_copy(
                        rows_v.at[lax.rem(j + 2, 3)],
                        out1_hbm.at[pl.ds(a0 + (q * SG + j - 1) * B, B),
                                    pl.ds(c * hc, hc)],
                        sem_w).wait()

                @pl.when(j + 2 < SG)
                def _():
                    pltpu.async_copy(
                        agg_sh.at[eid_v.at[j + 2]],
                        rows_v.at[lax.rem(j + 2, 3)], sem_g)

                pltpu.async_copy(
                    rows_v.at[cur],
                    out1_hbm.at[pl.ds(a0 + (q * SG + j) * B, B),
                                pl.ds(c * hc, hc)], sem_w)
                return 0
            lax.fori_loop(0, SG, _chunk, 0)
            pltpu.make_async_copy(
                rows_v.at[(SG - 1) % 3],
                out1_hbm.at[pl.ds(a0 + (q * SG + SG - 1) * B, B),
                            pl.ds(c * hc, hc)],
                sem_w).wait()
            return 0
        lax.fori_loop(0, nch // SG, _grp3, 0)
        _p3.__exit__(None, None, None)

    return sc_fn


def kernel(nodes_representations, hyperedge_arg_node_idxs,
           unq_hyperedge_type_reprs, hyperedge_type_name_unq_idxs,
           unq_hyperedge_arg_name_reprs, hyperedge_arg_name_unq_idxs,
           hyperedge_arg_to_edge_id, num_edges, W):
    n, h = nodes_representations.shape
    a = hyperedge_arg_node_idxs.shape[0]
    e = hyperedge_type_name_unq_idxs.shape[0]

    table = _matmul_split(nodes_representations, W).reshape(NC * n, h // NC)

    idx2 = 2 * hyperedge_arg_node_idxs.astype(jnp.int32)
    gidx = jnp.stack([idx2, idx2 + 1]).reshape(NC, a // B, B)
    eid2 = hyperedge_arg_to_edge_id.astype(jnp.int32).reshape(a // B, B)

    ones8 = jnp.ones((B, 8), jnp.float32)
    out1, out2 = _make_sc(n, h, a, e)(table, gidx, eid2, ones8)
    return out1, out2


# speedup vs baseline: 1.1553x; 1.0031x over previous
import functools

import jax
import jax.numpy as jnp
from jax import lax
from jax.experimental import pallas as pl
from jax.experimental.pallas import tpu as pltpu
from jax.experimental.pallas import tpu_sc as plsc

NC = 2
NS = 16
L = 16


def _mm_body(x_ref, w_ref, o_ref):
    o_ref[...] = lax.dot_general(x_ref[...], w_ref[...],
                                 (((1,), (1,)), ((), ())),
                                 preferred_element_type=jnp.float32)


def _matmul_split(nodes, w):
    n, h = nodes.shape
    bn = 2000
    return pl.pallas_call(
        _mm_body,
        grid=(n // bn,),
        in_specs=[
            pl.BlockSpec((bn, h), lambda i: (i, 0)),
            pl.BlockSpec((h, h), lambda i: (0, 0)),
        ],
        out_specs=pl.BlockSpec((bn, h), lambda i: (i, 0)),
        out_shape=jax.ShapeDtypeStruct((n, h), jnp.float32),
    )(nodes, w)


B = 125
SG = 32
EC = 125


@functools.cache
def _make_sc(n, h, a, e):
    hc = h // NC
    pw = a // NS
    nch = pw // B
    ept = e // NS
    mesh = plsc.VectorSubcoreMesh(core_axis_name="c", subcore_axis_name="s")

    @functools.partial(
        pl.kernel,
        out_type=[
            jax.ShapeDtypeStruct((a, h), jnp.float32),
            jax.ShapeDtypeStruct((e, h), jnp.float32),
        ],
        mesh=mesh,
        compiler_params=pltpu.CompilerParams(use_tc_tiling_on_sc=False, needs_layout_passes=False),
        scratch_types=[
            pltpu.VMEM((SG, B), jnp.int32),
            pltpu.VMEM((SG, B), jnp.int32),
            pltpu.VMEM((3, B, hc), jnp.float32),
            pltpu.VMEM((B, 8), jnp.float32),
            pltpu.VMEM((2 * EC, 8), jnp.float32),
            pltpu.VMEM_SHARED((e, hc), jnp.float32),
            pltpu.VMEM_SHARED((e, 8), jnp.float32),
            pltpu.SemaphoreType.DMA,
            pltpu.SemaphoreType.DMA,
            pltpu.SemaphoreType.DMA,
            pltpu.SemaphoreType.DMA,
        ],
    )
    def sc_fn(table_hbm, gidx_hbm, eid_hbm, ones_hbm, out1_hbm, out2_hbm,
              gidx_v, eid_v, rows_v, ones_v, dstg_v,
              agg_sh, deg_sh, sem_g, sem_s, sem_w, sem_d):
        c = lax.axis_index("c")
        s = lax.axis_index("s")
        e0 = s * ept
        a0 = s * pw

        pltpu.sync_copy(ones_hbm, ones_v)

        def _zero_row(i, _):
            z = jnp.zeros((L,), jnp.float32)
            for k in range(hc // L):
                rows_v[0, i, pl.ds(k * L, L)] = z
            return 0
        lax.fori_loop(0, B, _zero_row, 0)

        def _zero_sh(q, _):
            pltpu.sync_copy(rows_v.at[0],
                            agg_sh.at[pl.ds(e0 + q * B, B)])
            pltpu.sync_copy(rows_v.at[0].at[pl.ds(0, B), pl.ds(0, 8)],
                            deg_sh.at[pl.ds(e0 + q * B, B)])
            return 0
        lax.fori_loop(0, ept // B, _zero_sh, 0)
        plsc.subcore_barrier()

        def _grp1(q, _):
            r0 = s * nch + q * SG
            pltpu.sync_copy(gidx_hbm.at[c, pl.ds(r0, SG)], gidx_v)
            pltpu.sync_copy(eid_hbm.at[pl.ds(r0, SG)], eid_v)
            pltpu.async_copy(table_hbm.at[gidx_v.at[0]], rows_v.at[0], sem_g)
            pltpu.async_copy(table_hbm.at[gidx_v.at[1]], rows_v.at[1], sem_g)

            def _chunk(j, _):
                cur = lax.rem(j, 3)
                pltpu.make_async_copy(
                    table_hbm.at[gidx_v.at[j]], rows_v.at[cur], sem_g).wait()

                @pl.when(j >= 1)
                def _():
                    pltpu.make_async_copy(
                        rows_v.at[lax.rem(j + 2, 3)],
                        agg_sh.at[eid_v.at[j - 1]], sem_s).wait()

                @pl.when(j + 2 < SG)
                def _():
                    pltpu.async_copy(
                        table_hbm.at[gidx_v.at[j + 2]],
                        rows_v.at[lax.rem(j + 2, 3)], sem_g)

                pltpu.async_copy(
                    rows_v.at[cur], agg_sh.at[eid_v.at[j]], sem_s, add=True)
                pltpu.async_copy(ones_v, deg_sh.at[eid_v.at[j]], sem_d,
                                 add=True)
                return 0
            lax.fori_loop(0, SG, _chunk, 0)
            pltpu.make_async_copy(
                rows_v.at[(SG - 1) % 3], agg_sh.at[eid_v.at[SG - 1]],
                sem_s).wait()

            def _drain(j, _):
                pltpu.make_async_copy(
                    ones_v, deg_sh.at[eid_v.at[0]], sem_d).wait()
                return 0
            lax.fori_loop(0, SG, _drain, 0)
            return 0
        lax.fori_loop(0, nch // SG, _grp1, 0)
        plsc.subcore_barrier()

        ng2 = ept // EC

        def _stage2(q, buf):
            eq = e0 + q * EC
            pltpu.async_copy(agg_sh.at[pl.ds(eq, EC)], rows_v.at[buf].at[pl.ds(0, EC)], sem_g)
            pltpu.async_copy(deg_sh.at[pl.ds(eq, EC)], dstg_v.at[pl.ds(buf * EC, EC)], sem_g)

        def _unstage2(q, buf):
            eq = e0 + q * EC
            pltpu.make_async_copy(agg_sh.at[pl.ds(eq, EC)], rows_v.at[buf].at[pl.ds(0, EC)], sem_g).wait()
            pltpu.make_async_copy(deg_sh.at[pl.ds(eq, EC)], dstg_v.at[pl.ds(buf * EC, EC)], sem_g).wait()

        def _wrwait2(q, buf):
            eq = e0 + q * EC
            pltpu.make_async_copy(rows_v.at[buf].at[pl.ds(0, EC)], agg_sh.at[pl.ds(eq, EC)], sem_s).wait()
            pltpu.make_async_copy(rows_v.at[buf].at[pl.ds(0, EC)], out2_hbm.at[pl.ds(eq, EC), pl.ds(c * hc, hc)], sem_w).wait()

        _stage2(0, 0)

        def _grp2(q, _):
            eq = e0 + q * EC
            cur = lax.rem(q, 2)
            nxt = 1 - cur
            _unstage2(q, cur)

            @pl.when(q >= 1)
            def _():
                _wrwait2(q - 1, nxt)

            @pl.when(q + 1 < ng2)
            def _():
                _stage2(q + 1, nxt)

            def _scale_row(i, _):
                ridx = jnp.full((L,), cur * EC + i, jnp.int32)
                cidx = jnp.zeros((L,), jnp.int32)
                inv = 1.0 / plsc.load_gather(dstg_v, [ridx, cidx])
                for k in range(hc // L):
                    sl = pl.ds(k * L, L)
                    rows_v[cur, i, sl] = rows_v[cur, i, sl] * inv
                return 0
            lax.fori_loop(0, EC, _scale_row, 0)
            pltpu.async_copy(rows_v.at[cur].at[pl.ds(0, EC)], agg_sh.at[pl.ds(eq, EC)], sem_s)
            pltpu.async_copy(rows_v.at[cur].at[pl.ds(0, EC)], out2_hbm.at[pl.ds(eq, EC), pl.ds(c * hc, hc)], sem_w)
            return 0
        lax.fori_loop(0, ng2, _grp2, 0)
        _wrwait2(ng2 - 1, lax.rem(ng2 - 1, 2))
        plsc.subcore_barrier()

        def _grp3(q, _):
            pltpu.sync_copy(eid_hbm.at[pl.ds(s * nch + q * SG, SG)], eid_v)
            pltpu.async_copy(agg_sh.at[eid_v.at[0]], rows_v.at[0], sem_g)
            pltpu.async_copy(agg_sh.at[eid_v.at[1]], rows_v.at[1], sem_g)

            def _chunk(j, _):
                cur = lax.rem(j, 3)
                pltpu.make_async_copy(
                    agg_sh.at[eid_v.at[j]], rows_v.at[cur], sem_g).wait()

                @pl.when(j >= 1)
                def _():
                    pltpu.make_async_copy(
                        rows_v.at[lax.rem(j + 2, 3)],
                        out1_hbm.at[pl.ds(a0 + (q * SG + j - 1) * B, B),
                                    pl.ds(c * hc, hc)],
                        sem_w).wait()

                @pl.when(j + 2 < SG)
                def _():
                    pltpu.async_copy(
                        agg_sh.at[eid_v.at[j + 2]],
                        rows_v.at[lax.rem(j + 2, 3)], sem_g)

                pltpu.async_copy(
                    rows_v.at[cur],
                    out1_hbm.at[pl.ds(a0 + (q * SG + j) * B, B),
                                pl.ds(c * hc, hc)], sem_w)
                return 0
            lax.fori_loop(0, SG, _chunk, 0)
            pltpu.make_async_copy(
                rows_v.at[(SG - 1) % 3],
                out1_hbm.at[pl.ds(a0 + (q * SG + SG - 1) * B, B),
                            pl.ds(c * hc, hc)],
                sem_w).wait()
            return 0
        lax.fori_loop(0, nch // SG, _grp3, 0)

    return sc_fn


def kernel(nodes_representations, hyperedge_arg_node_idxs,
           unq_hyperedge_type_reprs, hyperedge_type_name_unq_idxs,
           unq_hyperedge_arg_name_reprs, hyperedge_arg_name_unq_idxs,
           hyperedge_arg_to_edge_id, num_edges, W):
    n, h = nodes_representations.shape
    a = hyperedge_arg_node_idxs.shape[0]
    e = hyperedge_type_name_unq_idxs.shape[0]

    table = _matmul_split(nodes_representations, W).reshape(NC * n, h // NC)

    idx2 = 2 * hyperedge_arg_node_idxs.astype(jnp.int32)
    gidx = jnp.stack([idx2, idx2 + 1]).reshape(NC, a // B, B)
    eid2 = hyperedge_arg_to_edge_id.astype(jnp.int32).reshape(a // B, B)

    ones8 = jnp.ones((B, 8), jnp.float32)
    out1, out2 = _make_sc(n, h, a, e)(table, gidx, eid2, ones8)
    return out1, out2


# speedup vs baseline: 1.1650x; 1.0084x over previous
import functools

import jax
import jax.numpy as jnp
from jax import lax
from jax.experimental import pallas as pl
from jax.experimental.pallas import tpu as pltpu
from jax.experimental.pallas import tpu_sc as plsc

NC = 2
NS = 16
L = 16


def _mm_body(x_ref, w_ref, o_ref):
    o_ref[...] = lax.dot_general(x_ref[...], w_ref[...],
                                 (((1,), (1,)), ((), ())),
                                 preferred_element_type=jnp.float32)


def _matmul_split(nodes, w):
    n, h = nodes.shape
    bn = 2000
    return pl.pallas_call(
        _mm_body,
        grid=(n // bn,),
        in_specs=[
            pl.BlockSpec((bn, h), lambda i: (i, 0)),
            pl.BlockSpec((h, h), lambda i: (0, 0)),
        ],
        out_specs=pl.BlockSpec((bn, h), lambda i: (i, 0)),
        out_shape=jax.ShapeDtypeStruct((n, h), jnp.float32),
    )(nodes, w)


B = 125
SG = 40
EC = 125


@functools.cache
def _make_sc(n, h, a, e):
    hc = h // NC
    pw = a // NS
    nch = pw // B
    ept = e // NS
    mesh = plsc.VectorSubcoreMesh(core_axis_name="c", subcore_axis_name="s")

    @functools.partial(
        pl.kernel,
        out_type=[
            jax.ShapeDtypeStruct((a, h), jnp.float32),
            jax.ShapeDtypeStruct((e, h), jnp.float32),
        ],
        mesh=mesh,
        compiler_params=pltpu.CompilerParams(use_tc_tiling_on_sc=False, needs_layout_passes=False),
        scratch_types=[
            pltpu.VMEM((SG, B), jnp.int32),
            pltpu.VMEM((SG, B), jnp.int32),
            pltpu.VMEM((3, B, hc), jnp.float32),
            pltpu.VMEM((B, 8), jnp.float32),
            pltpu.VMEM((2 * EC, 8), jnp.float32),
            pltpu.VMEM_SHARED((e, hc), jnp.float32),
            pltpu.VMEM_SHARED((e, 8), jnp.float32),
            pltpu.SemaphoreType.DMA,
            pltpu.SemaphoreType.DMA,
            pltpu.SemaphoreType.DMA,
            pltpu.SemaphoreType.DMA,
        ],
    )
    def sc_fn(table_hbm, gidx_hbm, eid_hbm, ones_hbm, out1_hbm, out2_hbm,
              gidx_v, eid_v, rows_v, ones_v, dstg_v,
              agg_sh, deg_sh, sem_g, sem_s, sem_w, sem_d):
        c = lax.axis_index("c")
        s = lax.axis_index("s")
        e0 = s * ept
        a0 = s * pw

        pltpu.sync_copy(ones_hbm, ones_v)

        def _zero_row(i, _):
            z = jnp.zeros((L,), jnp.float32)
            for k in range(hc // L):
                rows_v[0, i, pl.ds(k * L, L)] = z
            return 0
        lax.fori_loop(0, B, _zero_row, 0)

        def _zero_sh(q, _):
            pltpu.sync_copy(rows_v.at[0],
                            agg_sh.at[pl.ds(e0 + q * B, B)])
            pltpu.sync_copy(rows_v.at[0].at[pl.ds(0, B), pl.ds(0, 8)],
                            deg_sh.at[pl.ds(e0 + q * B, B)])
            return 0
        lax.fori_loop(0, ept // B, _zero_sh, 0)
        plsc.subcore_barrier()

        def _grp1(q, _):
            r0 = s * nch + q * SG
            pltpu.sync_copy(gidx_hbm.at[c, pl.ds(r0, SG)], gidx_v)
            pltpu.sync_copy(eid_hbm.at[pl.ds(r0, SG)], eid_v)
            pltpu.async_copy(table_hbm.at[gidx_v.at[0]], rows_v.at[0], sem_g)
            pltpu.async_copy(table_hbm.at[gidx_v.at[1]], rows_v.at[1], sem_g)

            def _chunk(j, _):
                cur = lax.rem(j, 3)
                pltpu.make_async_copy(
                    table_hbm.at[gidx_v.at[j]], rows_v.at[cur], sem_g).wait()

                @pl.when(j >= 1)
                def _():
                    pltpu.make_async_copy(
                        rows_v.at[lax.rem(j + 2, 3)],
                        agg_sh.at[eid_v.at[j - 1]], sem_s).wait()

                @pl.when(j + 2 < SG)
                def _():
                    pltpu.async_copy(
                        table_hbm.at[gidx_v.at[j + 2]],
                        rows_v.at[lax.rem(j + 2, 3)], sem_g)

                pltpu.async_copy(
                    rows_v.at[cur], agg_sh.at[eid_v.at[j]], sem_s, add=True)
                pltpu.async_copy(ones_v, deg_sh.at[eid_v.at[j]], sem_d,
                                 add=True)
                return 0
            lax.fori_loop(0, SG, _chunk, 0)
            pltpu.make_async_copy(
                rows_v.at[(SG - 1) % 3], agg_sh.at[eid_v.at[SG - 1]],
                sem_s).wait()

            def _drain(j, _):
                pltpu.make_async_copy(
                    ones_v, deg_sh.at[eid_v.at[0]], sem_d).wait()
                return 0
            lax.fori_loop(0, SG, _drain, 0)
            return 0
        lax.fori_loop(0, nch // SG, _grp1, 0)
        plsc.subcore_barrier()

        ng2 = ept // EC

        def _stage2(q, buf):
            eq = e0 + q * EC
            pltpu.async_copy(agg_sh.at[pl.ds(eq, EC)], rows_v.at[buf].at[pl.ds(0, EC)], sem_g)
            pltpu.async_copy(deg_sh.at[pl.ds(eq, EC)], dstg_v.at[pl.ds(buf * EC, EC)], sem_g)

        def _unstage2(q, buf):
            eq = e0 + q * EC
            pltpu.make_async_copy(agg_sh.at[pl.ds(eq, EC)], rows_v.at[buf].at[pl.ds(0, EC)], sem_g).wait()
            pltpu.make_async_copy(deg_sh.at[pl.ds(eq, EC)], dstg_v.at[pl.ds(buf * EC, EC)], sem_g).wait()

        def _wrwait2(q, buf):
            eq = e0 + q * EC
            pltpu.make_async_copy(rows_v.at[buf].at[pl.ds(0, EC)], agg_sh.at[pl.ds(eq, EC)], sem_s).wait()
            pltpu.make_async_copy(rows_v.at[buf].at[pl.ds(0, EC)], out2_hbm.at[pl.ds(eq, EC), pl.ds(c * hc, hc)], sem_w).wait()

        _stage2(0, 0)

        def _grp2(q, _):
            eq = e0 + q * EC
            cur = lax.rem(q, 2)
            nxt = 1 - cur
            _unstage2(q, cur)

            @pl.when(q >= 1)
            def _():
                _wrwait2(q - 1, nxt)

            @pl.when(q + 1 < ng2)
            def _():
                _stage2(q + 1, nxt)

            def _scale_row(i, _):
                ridx = jnp.full((L,), cur * EC + i, jnp.int32)
                cidx = jnp.zeros((L,), jnp.int32)
                inv = 1.0 / plsc.load_gather(dstg_v, [ridx, cidx])
                for k in range(hc // L):
                    sl = pl.ds(k * L, L)
                    rows_v[cur, i, sl] = rows_v[cur, i, sl] * inv
                return 0
            lax.fori_loop(0, EC, _scale_row, 0)
            pltpu.async_copy(rows_v.at[cur].at[pl.ds(0, EC)], agg_sh.at[pl.ds(eq, EC)], sem_s)
            pltpu.async_copy(rows_v.at[cur].at[pl.ds(0, EC)], out2_hbm.at[pl.ds(eq, EC), pl.ds(c * hc, hc)], sem_w)
            return 0
        lax.fori_loop(0, ng2, _grp2, 0)
        _wrwait2(ng2 - 1, lax.rem(ng2 - 1, 2))
        plsc.subcore_barrier()

        def _grp3(q, _):
            pltpu.sync_copy(eid_hbm.at[pl.ds(s * nch + q * SG, SG)], eid_v)
            pltpu.async_copy(agg_sh.at[eid_v.at[0]], rows_v.at[0], sem_g)
            pltpu.async_copy(agg_sh.at[eid_v.at[1]], rows_v.at[1], sem_g)

            def _chunk(j, _):
                cur = lax.rem(j, 3)
                pltpu.make_async_copy(
                    agg_sh.at[eid_v.at[j]], rows_v.at[cur], sem_g).wait()

                @pl.when(j >= 1)
                def _():
                    pltpu.make_async_copy(
                        rows_v.at[lax.rem(j + 2, 3)],
                        out1_hbm.at[pl.ds(a0 + (q * SG + j - 1) * B, B),
                                    pl.ds(c * hc, hc)],
                        sem_w).wait()

                @pl.when(j + 2 < SG)
                def _():
                    pltpu.async_copy(
                        agg_sh.at[eid_v.at[j + 2]],
                        rows_v.at[lax.rem(j + 2, 3)], sem_g)

                pltpu.async_copy(
                    rows_v.at[cur],
                    out1_hbm.at[pl.ds(a0 + (q * SG + j) * B, B),
                                pl.ds(c * hc, hc)], sem_w)
                return 0
            lax.fori_loop(0, SG, _chunk, 0)
            pltpu.make_async_copy(
                rows_v.at[(SG - 1) % 3],
                out1_hbm.at[pl.ds(a0 + (q * SG + SG - 1) * B, B),
                            pl.ds(c * hc, hc)],
                sem_w).wait()
            return 0
        lax.fori_loop(0, nch // SG, _grp3, 0)

    return sc_fn


def kernel(nodes_representations, hyperedge_arg_node_idxs,
           unq_hyperedge_type_reprs, hyperedge_type_name_unq_idxs,
           unq_hyperedge_arg_name_reprs, hyperedge_arg_name_unq_idxs,
           hyperedge_arg_to_edge_id, num_edges, W):
    n, h = nodes_representations.shape
    a = hyperedge_arg_node_idxs.shape[0]
    e = hyperedge_type_name_unq_idxs.shape[0]

    table = _matmul_split(nodes_representations, W).reshape(NC * n, h // NC)

    idx2 = 2 * hyperedge_arg_node_idxs.astype(jnp.int32)
    gidx = jnp.stack([idx2, idx2 + 1]).reshape(NC, a // B, B)
    eid2 = hyperedge_arg_to_edge_id.astype(jnp.int32).reshape(a // B, B)

    ones8 = jnp.ones((B, 8), jnp.float32)
    out1, out2 = _make_sc(n, h, a, e)(table, gidx, eid2, ones8)
    return out1, out2
